# baseline (speedup 1.0000x reference)
import functools

import jax
import jax.numpy as jnp
from jax import lax
from jax.experimental import pallas as pl
from jax.experimental.pallas import tpu as pltpu
from jax.experimental.pallas import tpu_sc as plsc

B = 4096
T = 200
C = 64
NC = 2
NS = 16
NW = NC * NS
ROWS_PER_W = B // NW
HALF = T // 2
LANES = 16
VECS_PER_ROW = C // LANES


def _body(idx_hbm, tok_hbm, pos_hbm, out_hbm, idx_v, pos_v, rows_v, sem0, sem1):
    cid = lax.axis_index("c")
    sid = lax.axis_index("s")
    w = sid * NC + cid

    pltpu.sync_copy(idx_hbm.at[w], idx_v)
    pltpu.sync_copy(pos_hbm, pos_v)

    def row_loop(r, carry):
        cp0 = pltpu.async_copy(
            tok_hbm.at[idx_v.at[2 * r]], rows_v.at[pl.ds(0, HALF)], sem0)
        cp1 = pltpu.async_copy(
            tok_hbm.at[idx_v.at[2 * r + 1]], rows_v.at[pl.ds(HALF, HALF)], sem1)
        cp0.wait()
        cp1.wait()

        def add_row(i, c2):
            for c in range(VECS_PER_ROW):
                sl = pl.ds(c * LANES, LANES)
                plsc.addupdate(rows_v.at[i, sl], pos_v[i, sl])
            return c2

        lax.fori_loop(0, T, add_row, 0)
        pltpu.sync_copy(rows_v, out_hbm.at[w * ROWS_PER_W + r])
        return carry

    lax.fori_loop(0, ROWS_PER_W, row_loop, 0)


@functools.partial(jax.jit, static_argnums=())
def _run(idx32, tok, pos):
    mesh = plsc.VectorSubcoreMesh(core_axis_name="c", subcore_axis_name="s")
    k = functools.partial(
        pl.kernel,
        mesh=mesh,
        out_type=jax.ShapeDtypeStruct((B, T, C), jnp.float32),
        scratch_types=[
            pltpu.VMEM((2 * ROWS_PER_W, HALF), jnp.int32),
            pltpu.VMEM((T, C), jnp.float32),
            pltpu.VMEM((T, C), jnp.float32),
            pltpu.SemaphoreType.DMA,
            pltpu.SemaphoreType.DMA,
        ],
        compiler_params=pltpu.CompilerParams(use_tc_tiling_on_sc=False),
    )(_body)
    return k(idx32, tok, pos)


def kernel(idx, token_embedding_table, position_embedding_table):
    idx32 = idx.astype(jnp.int32).reshape(NW, 2 * ROWS_PER_W, HALF)
    return _run(idx32, token_embedding_table, position_embedding_table)
